# baseline (speedup 1.0000x reference)
import functools

import jax
import jax.numpy as jnp
from jax import lax
from jax.experimental import pallas as pl
from jax.experimental.pallas import tpu as pltpu
from jax.experimental.pallas import tpu_sc as plsc


def _sc_gather_pairs(table2, idx2):
    info = plsc.get_sparse_core_info()
    nc, ns = info.num_cores, info.num_subcores
    nw = nc * ns
    b, d2 = idx2.shape[0], table2.shape[1]
    b_per_w = b // nw
    mesh = plsc.VectorSubcoreMesh(core_axis_name="c", subcore_axis_name="s")

    @functools.partial(
        pl.kernel,
        mesh=mesh,
        out_type=jax.ShapeDtypeStruct((b, d2), jnp.float32),
        scratch_types=[
            pltpu.VMEM((b_per_w,), jnp.int32),
            pltpu.VMEM((b_per_w, d2), jnp.float32),
            pltpu.SemaphoreType.DMA,
        ],
    )
    def gather_kernel(table_hbm, idx_hbm, out_hbm, idx_v, rows_v, sem):
        wid = lax.axis_index("s") * nc + lax.axis_index("c")
        base = wid * b_per_w
        pltpu.sync_copy(idx_hbm.at[pl.ds(base, b_per_w)], idx_v)
        pltpu.async_copy(table_hbm.at[idx_v], rows_v, sem).wait()
        pltpu.sync_copy(rows_v, out_hbm.at[pl.ds(base, b_per_w)])

    return gather_kernel(table2, idx2)


def _mm_body(pairs_ref, par_ref, w_ref, b_ref, o_ref, e_scr):
    d = e_scr.shape[1]

    @pl.when(pl.program_id(0) == 0)
    def _():
        e_scr[...] = jnp.where(
            par_ref[...] == 1, pairs_ref[:, d:], pairs_ref[:, :d]
        )

    o_ref[...] = (
        lax.dot_general(
            e_scr[...],
            w_ref[...],
            (((1,), (1,)), ((), ())),
            preferred_element_type=jnp.float32,
        )
        + b_ref[...]
    )


def _tc_project(pairs, parity, W, b, vblk=1024):
    bsz, d2 = pairs.shape
    d = d2 // 2
    vocab = W.shape[0]
    nv = pl.cdiv(vocab, vblk)
    return pl.pallas_call(
        _mm_body,
        grid=(nv,),
        in_specs=[
            pl.BlockSpec((bsz, d2), lambda i: (0, 0)),
            pl.BlockSpec((bsz, 1), lambda i: (0, 0)),
            pl.BlockSpec((vblk, d), lambda i: (i, 0)),
            pl.BlockSpec((1, vblk), lambda i: (0, i)),
        ],
        out_specs=pl.BlockSpec((bsz, vblk), lambda i: (0, i)),
        out_shape=jax.ShapeDtypeStruct((bsz, vocab), jnp.float32),
        scratch_shapes=[pltpu.VMEM((bsz, d), jnp.float32)],
    )(pairs, parity, W, b.reshape(1, vocab))


def kernel(inputs, table, W, b):
    vocab, d = table.shape
    table2 = table.reshape(vocab // 2, 2 * d)
    pairs = _sc_gather_pairs(table2, inputs >> 1)
    parity = (inputs & 1).reshape(inputs.shape[0], 1)
    return _tc_project(pairs, parity, W, b)


# speedup vs baseline: 1.0178x; 1.0178x over previous
import functools

import jax
import jax.numpy as jnp
from jax import lax
from jax.experimental import pallas as pl
from jax.experimental.pallas import tpu as pltpu
from jax.experimental.pallas import tpu_sc as plsc

_VBLK = 1024
_NBUF = 6


def _sc_gather_pairs(table2, idx2):
    info = plsc.get_sparse_core_info()
    nc, ns = info.num_cores, info.num_subcores
    nw = nc * ns
    b, d2 = idx2.shape[0], table2.shape[1]
    b_per_w = b // nw
    mesh = plsc.VectorSubcoreMesh(core_axis_name="c", subcore_axis_name="s")

    @functools.partial(
        pl.kernel,
        mesh=mesh,
        out_type=jax.ShapeDtypeStruct((b, d2), jnp.float32),
        scratch_types=[
            pltpu.VMEM((b_per_w,), jnp.int32),
            pltpu.VMEM((b_per_w, d2), jnp.float32),
            pltpu.SemaphoreType.DMA,
        ],
    )
    def gather_kernel(table_hbm, idx_hbm, out_hbm, idx_v, rows_v, sem):
        wid = lax.axis_index("s") * nc + lax.axis_index("c")
        base = wid * b_per_w
        pltpu.sync_copy(idx_hbm.at[pl.ds(base, b_per_w)], idx_v)
        pltpu.async_copy(table_hbm.at[idx_v], rows_v, sem).wait()
        pltpu.sync_copy(rows_v, out_hbm.at[pl.ds(base, b_per_w)])

    return gather_kernel(table2, idx2)


def _mm_body(
    pairs_ref, par_ref, w_ref, b_ref, wt_ref, bt_ref, o_hbm,
    e_scr, obuf, tbuf, sems, tsem,
):
    i = pl.program_id(0)
    ng = pl.num_programs(0)
    d = e_scr.shape[1]
    slot = lax.rem(i, _NBUF)

    @pl.when(i == 0)
    def _():
        e_scr[...] = jnp.where(
            par_ref[...] == 1, pairs_ref[:, d:], pairs_ref[:, :d]
        )

    @pl.when(i >= _NBUF)
    def _():
        pltpu.make_async_copy(
            obuf.at[slot], o_hbm.at[:, pl.ds(0, _VBLK)], sems.at[slot]
        ).wait()

    obuf[slot] = (
        lax.dot_general(
            e_scr[...],
            w_ref[...],
            (((1,), (1,)), ((), ())),
            preferred_element_type=jnp.float32,
        )
        + b_ref[0]
    )
    pltpu.make_async_copy(
        obuf.at[slot], o_hbm.at[:, pl.ds(i * _VBLK, _VBLK)], sems.at[slot]
    ).start()

    @pl.when(i == ng - 1)
    def _():
        tail = bt_ref.shape[1]
        tbuf[...] = (
            lax.dot_general(
                e_scr[...],
                wt_ref[...],
                (((1,), (1,)), ((), ())),
                preferred_element_type=jnp.float32,
            )
            + bt_ref[...]
        )
        pltpu.make_async_copy(
            tbuf, o_hbm.at[:, pl.ds(ng * _VBLK, tail)], tsem
        ).start()
        pltpu.make_async_copy(
            tbuf, o_hbm.at[:, pl.ds(ng * _VBLK, tail)], tsem
        ).wait()
        for s in range(_NBUF):
            pltpu.make_async_copy(
                obuf.at[s], o_hbm.at[:, pl.ds(0, _VBLK)], sems.at[s]
            ).wait()


def _tc_project(pairs, parity, W, b):
    bsz, d2 = pairs.shape
    d = d2 // 2
    vocab = W.shape[0]
    ng = vocab // _VBLK
    main = ng * _VBLK
    tail = vocab - main
    b3 = b[:main].reshape(ng, 1, _VBLK)
    wt = W[main:]
    bt = b[main:].reshape(1, tail)
    return pl.pallas_call(
        _mm_body,
        grid=(ng,),
        in_specs=[
            pl.BlockSpec((bsz, d2), lambda i: (0, 0)),
            pl.BlockSpec((bsz, 1), lambda i: (0, 0)),
            pl.BlockSpec((_VBLK, d), lambda i: (i, 0)),
            pl.BlockSpec((1, 1, _VBLK), lambda i: (i, 0, 0)),
            pl.BlockSpec((tail, d), lambda i: (0, 0)),
            pl.BlockSpec((1, tail), lambda i: (0, 0)),
        ],
        out_specs=pl.BlockSpec(memory_space=pl.ANY),
        out_shape=jax.ShapeDtypeStruct((bsz, vocab), jnp.float32),
        scratch_shapes=[
            pltpu.VMEM((bsz, d), jnp.float32),
            pltpu.VMEM((_NBUF, bsz, _VBLK), jnp.float32),
            pltpu.VMEM((bsz, tail), jnp.float32),
            pltpu.SemaphoreType.DMA((_NBUF,)),
            pltpu.SemaphoreType.DMA,
        ],
    )(pairs, parity, W, b3, wt, bt)


def kernel(inputs, table, W, b):
    vocab, d = table.shape
    table2 = table.reshape(vocab // 2, 2 * d)
    pairs = _sc_gather_pairs(table2, inputs >> 1)
    parity = (inputs & 1).reshape(inputs.shape[0], 1)
    return _tc_project(pairs, parity, W, b)


# speedup vs baseline: 1.0191x; 1.0012x over previous
import functools

import jax
import jax.numpy as jnp
from jax import lax
from jax.experimental import pallas as pl
from jax.experimental.pallas import tpu as pltpu
from jax.experimental.pallas import tpu_sc as plsc

_VBLK = 1024
_NBUF = 6


def _sc_gather_pairs(table2, idx2):
    info = plsc.get_sparse_core_info()
    nc, ns = info.num_cores, info.num_subcores
    nw = nc * ns
    b, d2 = idx2.shape[0], table2.shape[1]
    b_per_w = b // nw
    mesh = plsc.VectorSubcoreMesh(core_axis_name="c", subcore_axis_name="s")

    @functools.partial(
        pl.kernel,
        mesh=mesh,
        out_type=jax.ShapeDtypeStruct((b, d2), jnp.float32),
        scratch_types=[
            pltpu.VMEM((b_per_w,), jnp.int32),
            pltpu.VMEM((b_per_w, d2), jnp.float32),
            pltpu.SemaphoreType.DMA,
        ],
    )
    def gather_kernel(table_hbm, idx_hbm, out_hbm, idx_v, rows_v, sem):
        wid = lax.axis_index("s") * nc + lax.axis_index("c")
        base = wid * b_per_w
        pltpu.sync_copy(idx_hbm.at[pl.ds(base, b_per_w)], idx_v)
        pltpu.async_copy(table_hbm.at[idx_v], rows_v, sem).wait()
        pltpu.sync_copy(rows_v, out_hbm.at[pl.ds(base, b_per_w)])

    return gather_kernel(table2, idx2)


def _mm_body(
    pairs_ref, par_ref, w_ref, b_ref, wt_ref, bt_ref, o_hbm,
    e_scr, obuf, tbuf, sems, tsem,
):
    i = pl.program_id(0)
    ng = pl.num_programs(0)
    d = e_scr.shape[1]
    slot = lax.rem(i, _NBUF)

    @pl.when(i == 0)
    def _():
        e_scr[...] = jnp.where(
            par_ref[...] == 1, pairs_ref[:, d:], pairs_ref[:, :d]
        )

    @pl.when(i >= _NBUF)
    def _():
        pltpu.make_async_copy(
            obuf.at[slot], o_hbm.at[:, pl.ds(0, _VBLK)], sems.at[slot]
        ).wait()

    obuf[slot] = (
        lax.dot_general(
            e_scr[...],
            w_ref[...],
            (((1,), (1,)), ((), ())),
            preferred_element_type=jnp.float32,
        )
        + b_ref[0]
    )
    for s in range(_NBUF):

        @pl.when(slot == s)
        def _(s=s):
            pltpu.make_async_copy(
                obuf.at[s], o_hbm.at[:, pl.ds(i * _VBLK, _VBLK)], sems.at[s]
            ).start(priority=s % 2)

    @pl.when(i == ng - 1)
    def _():
        tail = bt_ref.shape[1]
        tbuf[...] = (
            lax.dot_general(
                e_scr[...],
                wt_ref[...],
                (((1,), (1,)), ((), ())),
                preferred_element_type=jnp.float32,
            )
            + bt_ref[...]
        )
        pltpu.make_async_copy(
            tbuf, o_hbm.at[:, pl.ds(ng * _VBLK, tail)], tsem
        ).start()
        pltpu.make_async_copy(
            tbuf, o_hbm.at[:, pl.ds(ng * _VBLK, tail)], tsem
        ).wait()
        for s in range(_NBUF):
            pltpu.make_async_copy(
                obuf.at[s], o_hbm.at[:, pl.ds(0, _VBLK)], sems.at[s]
            ).wait()


def _tc_project(pairs, parity, W, b):
    bsz, d2 = pairs.shape
    d = d2 // 2
    vocab = W.shape[0]
    ng = vocab // _VBLK
    main = ng * _VBLK
    tail = vocab - main
    b3 = b[:main].reshape(ng, 1, _VBLK)
    wt = W[main:]
    bt = b[main:].reshape(1, tail)
    return pl.pallas_call(
        _mm_body,
        grid=(ng,),
        in_specs=[
            pl.BlockSpec((bsz, d2), lambda i: (0, 0)),
            pl.BlockSpec((bsz, 1), lambda i: (0, 0)),
            pl.BlockSpec((_VBLK, d), lambda i: (i, 0)),
            pl.BlockSpec((1, 1, _VBLK), lambda i: (i, 0, 0)),
            pl.BlockSpec((tail, d), lambda i: (0, 0)),
            pl.BlockSpec((1, tail), lambda i: (0, 0)),
        ],
        out_specs=pl.BlockSpec(memory_space=pl.ANY),
        out_shape=jax.ShapeDtypeStruct((bsz, vocab), jnp.float32),
        scratch_shapes=[
            pltpu.VMEM((bsz, d), jnp.float32),
            pltpu.VMEM((_NBUF, bsz, _VBLK), jnp.float32),
            pltpu.VMEM((bsz, tail), jnp.float32),
            pltpu.SemaphoreType.DMA((_NBUF,)),
            pltpu.SemaphoreType.DMA,
        ],
    )(pairs, parity, W, b3, wt, bt)


def kernel(inputs, table, W, b):
    vocab, d = table.shape
    table2 = table.reshape(vocab // 2, 2 * d)
    pairs = _sc_gather_pairs(table2, inputs >> 1)
    parity = (inputs & 1).reshape(inputs.shape[0], 1)
    return _tc_project(pairs, parity, W, b)
